# baseline (speedup 1.0000x reference)
import functools

import jax
import jax.numpy as jnp
from jax import lax
from jax.experimental import pallas as pl
from jax.experimental.pallas import tpu as pltpu
from jax.experimental.pallas import tpu_sc as plsc

_B = 16384
_L = 50
_EMB = 64
_TOTAL = _B * _L
_NC = 2
_NS = 16
_NW = _NC * _NS
_BB = _B // 128
_NP = _L * _BB
_PPW = _NP // _NW
_PER_W = _PPW * 128
_NG = 4
_NO = 2


@jax.jit
def _gather_call(table, idx):
    mesh = plsc.VectorSubcoreMesh(core_axis_name="c", subcore_axis_name="s")

    @functools.partial(
        pl.kernel,
        mesh=mesh,
        out_type=jax.ShapeDtypeStruct((_L, _EMB // 8, _BB, 8, 128),
                                      jnp.float32),
        scratch_types=(
            [pltpu.VMEM((_PER_W,), jnp.int32)]
            + [pltpu.VMEM((128, _EMB), jnp.float32)] * _NG
            + [pltpu.VMEM((_EMB // 8, 8, 128), jnp.float32)] * _NO
            + [pltpu.SemaphoreType.DMA] * (_NG + _NO)
        ),
        compiler_params=pltpu.CompilerParams(use_tc_tiling_on_sc=False,
                                             needs_layout_passes=False),
    )
    def k(table_hbm, idx_hbm, out_hbm, idx_v, *bufs):
        rows = bufs[:_NG]
        trows = bufs[_NG:_NG + _NO]
        gsem = bufs[_NG + _NO:2 * _NG + _NO]
        osem = bufs[2 * _NG + _NO:]
        wid = lax.axis_index("s") * _NC + lax.axis_index("c")
        p0 = wid * _PPW
        pltpu.sync_copy(idx_hbm.at[pl.ds(p0 * 128, _PER_W)], idx_v)
        biota = lax.iota(jnp.int32, 16)

        def start_gather(j, g):
            pltpu.async_copy(table_hbm.at[idx_v.at[pl.ds(j * 128, 128)]],
                             rows[g], gsem[g])

        def wait_gather(j, g):
            pltpu.make_async_copy(table_hbm.at[idx_v.at[pl.ds(j * 128, 128)]],
                                  rows[g], gsem[g]).wait()

        def transpose(g, o):
            def erow(eb, c):
                for ei in range(8):
                    es = jnp.full((16,), eb * 8 + ei, jnp.int32)
                    for bc in range(8):
                        v = plsc.load_gather(rows[g], [biota + bc * 16, es])
                        trows[o][eb, ei, pl.ds(bc * 16, 16)] = v
                return c
            lax.fori_loop(0, _EMB // 8, erow, 0)

        def start_store(j, o):
            p = p0 + j
            l = p // _BB
            bb = p % _BB
            for eb in range(_EMB // 8):
                pltpu.async_copy(trows[o].at[eb], out_hbm.at[l, eb, bb],
                                 osem[o])

        def wait_store(j, o):
            p = p0 + j
            l = p // _BB
            bb = p % _BB
            for eb in range(_EMB // 8):
                pltpu.make_async_copy(trows[o].at[eb], out_hbm.at[l, eb, bb],
                                      osem[o]).wait()

        for j in range(_NG):
            start_gather(j, j)

        def body(i, carry):
            for g in range(_NG):
                j = i * _NG + g
                o = g % _NO
                wait_gather(j, g)

                @pl.when(j >= _NO)
                def _():
                    wait_store(j - _NO, o)

                transpose(g, o)
                start_store(j, o)

                @pl.when(j + _NG < _PPW)
                def _():
                    start_gather(j + _NG, g)
            return carry

        lax.fori_loop(0, _PPW // _NG, body, 0)
        for j in range(_PPW - _NO, _PPW):
            wait_store(j, j % _NO)

    return k(table, idx)


def kernel(data, iword_indicator, iword_numerals, ivectors_weight,
           gmm_posterior, iprototypes_embeddings):
    idx = data.T.reshape(_TOTAL)
    out5 = _gather_call(ivectors_weight, idx)
    return out5.transpose(2, 4, 0, 1, 3).reshape(_B, _L, _EMB)


# speedup vs baseline: 1.1898x; 1.1898x over previous
import functools

import jax
import jax.numpy as jnp
from jax import lax
from jax.experimental import pallas as pl
from jax.experimental.pallas import tpu as pltpu
from jax.experimental.pallas import tpu_sc as plsc

_B = 16384
_L = 50
_EMB = 64
_TOTAL = _B * _L
_NC = 2
_NS = 16
_NW = _NC * _NS
_BB = _B // 128
_NP = _L * _BB
_PPW = _NP // _NW
_PER_W = _PPW * 128
_NG = 4
_NO = 2


@jax.jit
def _gather_call(table, idx):
    mesh = plsc.VectorSubcoreMesh(core_axis_name="c", subcore_axis_name="s")

    @functools.partial(
        pl.kernel,
        mesh=mesh,
        out_type=jax.ShapeDtypeStruct((_L, _EMB // 8, _BB, 8, 128),
                                      jnp.float32),
        scratch_types=(
            [pltpu.VMEM((_PER_W,), jnp.int32)]
            + [pltpu.VMEM((128, _EMB), jnp.float32)] * _NG
            + [pltpu.VMEM((_EMB, 128), jnp.float32)] * _NO
            + [pltpu.SemaphoreType.DMA] * (_NG + _NO)
        ),
        compiler_params=pltpu.CompilerParams(use_tc_tiling_on_sc=False,
                                             needs_layout_passes=False),
    )
    def k(table_hbm, idx_hbm, out_hbm, idx_v, *bufs):
        rows = bufs[:_NG]
        trows = bufs[_NG:_NG + _NO]
        gsem = bufs[_NG + _NO:2 * _NG + _NO]
        osem = bufs[2 * _NG + _NO:]
        wid = lax.axis_index("s") * _NC + lax.axis_index("c")
        p0 = wid * _PPW
        pltpu.sync_copy(idx_hbm.at[pl.ds(p0 * 128, _PER_W)], idx_v)
        biota = lax.iota(jnp.int32, 16)

        def start_gather(j, g):
            pltpu.async_copy(table_hbm.at[idx_v.at[pl.ds(j * 128, 128)]],
                             rows[g], gsem[g])

        def wait_gather(j, g):
            pltpu.make_async_copy(table_hbm.at[idx_v.at[pl.ds(j * 128, 128)]],
                                  rows[g], gsem[g]).wait()

        eidx = [biota + c4 * 16 for c4 in range(_EMB // 16)]

        def transpose(g, o):
            def brow(b, c):
                bs = jnp.full((16,), b, jnp.int32)
                for c4 in range(_EMB // 16):
                    v = rows[g][b, pl.ds(c4 * 16, 16)]
                    plsc.store_scatter(trows[o], [eidx[c4], bs], v)
                return c
            lax.fori_loop(0, 128, brow, 0)

        def start_store(j, o):
            p = p0 + j
            l = p // _BB
            bb = p % _BB
            for eb in range(_EMB // 8):
                pltpu.async_copy(trows[o].at[pl.ds(eb * 8, 8)],
                                 out_hbm.at[l, eb, bb], osem[o])

        def wait_store(j, o):
            p = p0 + j
            l = p // _BB
            bb = p % _BB
            for eb in range(_EMB // 8):
                pltpu.make_async_copy(trows[o].at[pl.ds(eb * 8, 8)],
                                      out_hbm.at[l, eb, bb], osem[o]).wait()

        for j in range(_NG):
            start_gather(j, j)

        def body(i, carry):
            for g in range(_NG):
                j = i * _NG + g
                o = g % _NO
                wait_gather(j, g)

                @pl.when(j >= _NO)
                def _():
                    wait_store(j - _NO, o)

                transpose(g, o)
                start_store(j, o)

                @pl.when(j + _NG < _PPW)
                def _():
                    start_gather(j + _NG, g)
            return carry

        lax.fori_loop(0, _PPW // _NG, body, 0)
        for j in range(_PPW - _NO, _PPW):
            wait_store(j, j % _NO)

    return k(table, idx)


def kernel(data, iword_indicator, iword_numerals, ivectors_weight,
           gmm_posterior, iprototypes_embeddings):
    idx = data.T.reshape(_TOTAL)
    out5 = _gather_call(ivectors_weight, idx)
    return out5.transpose(2, 4, 0, 1, 3).reshape(_B, _L, _EMB)


# speedup vs baseline: 1.2304x; 1.0341x over previous
import functools

import jax
import jax.numpy as jnp
from jax import lax
from jax.experimental import pallas as pl
from jax.experimental.pallas import tpu as pltpu
from jax.experimental.pallas import tpu_sc as plsc

_B = 16384
_L = 50
_EMB = 64
_TOTAL = _B * _L
_NC = 2
_NS = 16
_NW = _NC * _NS
_BB = _B // 128
_NP = _L * _BB
_PPW = _NP // _NW
_PER_W = _PPW * 128
_NG = 4
_NO = 2


@jax.jit
def _gather_call(table, idx):
    mesh = plsc.VectorSubcoreMesh(core_axis_name="c", subcore_axis_name="s")

    @functools.partial(
        pl.kernel,
        mesh=mesh,
        out_type=jax.ShapeDtypeStruct((_L, _EMB // 8, _BB, 8, 128),
                                      jnp.float32),
        scratch_types=(
            [pltpu.VMEM((_PER_W,), jnp.int32)]
            + [pltpu.VMEM((128, _EMB), jnp.float32)] * _NG
            + [pltpu.VMEM((_EMB, 128), jnp.float32)] * _NO
            + [pltpu.SemaphoreType.DMA] * (_NG + _NO)
        ),
        compiler_params=pltpu.CompilerParams(use_tc_tiling_on_sc=False,
                                             needs_layout_passes=False),
    )
    def k(table_hbm, idx_hbm, out_hbm, idx_v, *bufs):
        rows = bufs[:_NG]
        trows = bufs[_NG:_NG + _NO]
        gsem = bufs[_NG + _NO:2 * _NG + _NO]
        osem = bufs[2 * _NG + _NO:]
        wid = lax.axis_index("s") * _NC + lax.axis_index("c")
        p0 = wid * _PPW
        pltpu.sync_copy(idx_hbm.at[pl.ds(p0 * 128, _PER_W)], idx_v)
        biota = lax.iota(jnp.int32, 16)

        def start_gather(j, g):
            pltpu.async_copy(table_hbm.at[idx_v.at[pl.ds(j * 128, 128)]],
                             rows[g], gsem[g])

        def wait_gather(j, g):
            pltpu.make_async_copy(table_hbm.at[idx_v.at[pl.ds(j * 128, 128)]],
                                  rows[g], gsem[g]).wait()

        eidx = [biota + c4 * 16 for c4 in range(_EMB // 16)]

        def transpose(g, o):
            def brow(b0, c):
                vs = []
                for k in range(8):
                    b = b0 * 8 + k
                    for c4 in range(_EMB // 16):
                        vs.append(rows[g][b, pl.ds(c4 * 16, 16)])
                for k in range(8):
                    b = b0 * 8 + k
                    bs = jnp.full((16,), b, jnp.int32)
                    for c4 in range(_EMB // 16):
                        plsc.store_scatter(trows[o], [eidx[c4], bs],
                                           vs[k * (_EMB // 16) + c4])
                return c
            lax.fori_loop(0, 16, brow, 0)

        def start_store(j, o):
            p = p0 + j
            l = p // _BB
            bb = p % _BB
            for eb in range(_EMB // 8):
                pltpu.async_copy(trows[o].at[pl.ds(eb * 8, 8)],
                                 out_hbm.at[l, eb, bb], osem[o])

        def wait_store(j, o):
            p = p0 + j
            l = p // _BB
            bb = p % _BB
            for eb in range(_EMB // 8):
                pltpu.make_async_copy(trows[o].at[pl.ds(eb * 8, 8)],
                                      out_hbm.at[l, eb, bb], osem[o]).wait()

        for j in range(_NG):
            start_gather(j, j)

        def body(i, carry):
            for g in range(_NG):
                j = i * _NG + g
                o = g % _NO
                wait_gather(j, g)

                @pl.when(j >= _NO)
                def _():
                    wait_store(j - _NO, o)

                transpose(g, o)
                start_store(j, o)

                @pl.when(j + _NG < _PPW)
                def _():
                    start_gather(j + _NG, g)
            return carry

        lax.fori_loop(0, _PPW // _NG, body, 0)
        for j in range(_PPW - _NO, _PPW):
            wait_store(j, j % _NO)

    return k(table, idx)


def kernel(data, iword_indicator, iword_numerals, ivectors_weight,
           gmm_posterior, iprototypes_embeddings):
    idx = data.T.reshape(_TOTAL)
    out5 = _gather_call(ivectors_weight, idx)
    return out5.transpose(2, 4, 0, 1, 3).reshape(_B, _L, _EMB)


# speedup vs baseline: 1.5156x; 1.2318x over previous
import functools

import jax
import jax.numpy as jnp
from jax import lax
from jax.experimental import pallas as pl
from jax.experimental.pallas import tpu as pltpu
from jax.experimental.pallas import tpu_sc as plsc

_B = 16384
_L = 50
_EMB = 64
_TOTAL = _B * _L
_NC = 2
_NS = 16
_NW = _NC * _NS
_PER_W = _TOTAL // _NW
_CHUNK = 256
_NCH = _PER_W // _CHUNK
_NBUF = 4


@jax.jit
def _gather_call2(table, idx):
    mesh = plsc.VectorSubcoreMesh(core_axis_name="c", subcore_axis_name="s")

    @functools.partial(
        pl.kernel,
        mesh=mesh,
        out_type=jax.ShapeDtypeStruct((_TOTAL, _EMB), jnp.float32),
        scratch_types=(
            [pltpu.VMEM((_PER_W,), jnp.int32)]
            + [pltpu.VMEM((_CHUNK, _EMB), jnp.float32)] * _NBUF
            + [pltpu.SemaphoreType.DMA] * (2 * _NBUF)
        ),
        compiler_params=pltpu.CompilerParams(use_tc_tiling_on_sc=False),
    )
    def k(table_hbm, idx_hbm, out_hbm, idx_v, *bufs):
        rows = bufs[:_NBUF]
        gsem = bufs[_NBUF:2 * _NBUF]
        osem = bufs[2 * _NBUF:]
        wid = lax.axis_index("s") * _NC + lax.axis_index("c")
        base = wid * _PER_W
        pltpu.sync_copy(idx_hbm.at[pl.ds(base, _PER_W)], idx_v)

        def start_gather(j, b):
            pltpu.async_copy(table_hbm.at[idx_v.at[pl.ds(j * _CHUNK, _CHUNK)]],
                             rows[b], gsem[b])

        def wait_gather(j, b):
            pltpu.make_async_copy(table_hbm.at[idx_v.at[pl.ds(j * _CHUNK, _CHUNK)]],
                                  rows[b], gsem[b]).wait()

        def out_slice(j):
            return out_hbm.at[pl.ds(base + j * _CHUNK, _CHUNK)]

        def start_store(j, b):
            pltpu.async_copy(rows[b], out_slice(j), osem[b])

        def wait_store(j, b):
            pltpu.make_async_copy(rows[b], out_slice(j), osem[b]).wait()

        for j in range(_NBUF - 1):
            start_gather(j, j)

        def body(i, carry):
            for b in range(_NBUF):
                j = i * _NBUF + b
                prv = (b - 1) % _NBUF

                @pl.when(j + _NBUF - 1 < _NCH)
                def _():
                    @pl.when(j >= 1)
                    def _():
                        wait_store(j - 1, prv)
                    start_gather(j + _NBUF - 1, prv)

                wait_gather(j, b)
                start_store(j, b)
            return carry

        lax.fori_loop(0, _NCH // _NBUF, body, 0)
        for j in range(_NCH - _NBUF, _NCH):
            wait_store(j, j % _NBUF)

    return k(table, idx)


def kernel(data, iword_indicator, iword_numerals, ivectors_weight,
           gmm_posterior, iprototypes_embeddings):
    idx = data.reshape(_TOTAL)
    out = _gather_call2(ivectors_weight, idx)
    return out.reshape(_B, _L, _EMB)
